# baseline (speedup 1.0000x reference)
import jax
import jax.numpy as jnp
from jax.experimental import pallas as pl
from jax.experimental.pallas import tpu as pltpu

_RADIUS = 1.0
_MAX_K = 64
_MIN_D = 20.0
_MAX_D = 60.0


def _mlp_kernel(w2_hbm, pk_hbm, b3_smem, out_ref, w2_v, pk_v, sem):
    copies = [
        pltpu.make_async_copy(w2_hbm, w2_v, sem),
        pltpu.make_async_copy(pk_hbm, pk_v, sem),
    ]
    for c in copies:
        c.start()
    for c in copies:
        c.wait()

    vol = 4.0 / 3.0 * 3.14159 * _RADIUS**3
    d_mean = jnp.float32(_MAX_K) / jnp.float32(vol)

    w1 = pk_v[0:1, :]
    b1 = pk_v[1:2, :]
    b2 = pk_v[2:3, :]
    w3 = pk_v[3:4, :]
    h1 = jnp.maximum(d_mean * w1 + b1, 0.0)
    h2 = jax.lax.dot_general(
        h1, w2_v[...], (((1,), (1,)), ((), ())),
        preferred_element_type=jnp.float32)
    h2 = jnp.maximum(h2 + b2, 0.0)
    z = jnp.sum(h2 * w3, axis=-1, keepdims=True) + b3_smem[0]
    t = jax.nn.sigmoid(z)
    thr = _MIN_D + (_MAX_D - _MIN_D) * t
    out_ref[...] = jnp.broadcast_to(thr, out_ref.shape)


def kernel(xyz, W1, b1, W2, b2, W3, b3):
    B = xyz.shape[0]
    packed = jnp.concatenate(
        [W1.reshape(1, -1), b1.reshape(1, -1), b2.reshape(1, -1), W3], axis=0)
    hbm = pl.BlockSpec(memory_space=pltpu.MemorySpace.HBM)
    out = pl.pallas_call(
        _mlp_kernel,
        out_shape=jax.ShapeDtypeStruct((1, B), jnp.float32),
        in_specs=[hbm, hbm, pl.BlockSpec(memory_space=pltpu.SMEM)],
        scratch_shapes=[
            pltpu.VMEM((64, 64), jnp.float32),
            pltpu.VMEM((4, 64), jnp.float32),
            pltpu.SemaphoreType.DMA,
        ],
    )(W2, packed, b3)
    return out.reshape(B)


# speedup vs baseline: 1.5706x; 1.5706x over previous
import jax
import jax.numpy as jnp
from jax.experimental import pallas as pl
from jax.experimental.pallas import tpu as pltpu

_RADIUS = 1.0
_MAX_K = 64
_MIN_D = 20.0
_MAX_D = 60.0


def _mlp_kernel(w1_hbm, b1_hbm, w2_hbm, b2_hbm, w3_hbm, b3_smem, out_ref,
                w1_v, b1_v, w2_v, b2_v, w3_v, sem):
    c_w1 = pltpu.make_async_copy(w1_hbm, w1_v, sem)
    c_b1 = pltpu.make_async_copy(b1_hbm, b1_v, sem)
    c_w2 = pltpu.make_async_copy(w2_hbm, w2_v, sem)
    c_b2 = pltpu.make_async_copy(b2_hbm, b2_v, sem)
    c_w3 = pltpu.make_async_copy(w3_hbm, w3_v, sem)
    for c in (c_w1, c_b1, c_w2, c_b2, c_w3):
        c.start()

    vol = 4.0 / 3.0 * 3.14159 * _RADIUS**3
    d_mean = jnp.float32(_MAX_K) / jnp.float32(vol)

    c_w1.wait()
    c_b1.wait()
    h1 = jnp.maximum(d_mean * w1_v[...] + b1_v[...], 0.0)
    c_w2.wait()
    h2 = jax.lax.dot_general(
        h1, w2_v[...], (((1,), (1,)), ((), ())),
        preferred_element_type=jnp.float32)
    c_b2.wait()
    h2 = jnp.maximum(h2 + b2_v[...], 0.0)
    c_w3.wait()
    z = jnp.sum(h2 * w3_v[...], axis=-1, keepdims=True) + b3_smem[0]
    t = jax.nn.sigmoid(z)
    thr = _MIN_D + (_MAX_D - _MIN_D) * t
    out_ref[...] = jnp.broadcast_to(thr, out_ref.shape)


def kernel(xyz, W1, b1, W2, b2, W3, b3):
    B = xyz.shape[0]
    hbm = pl.BlockSpec(memory_space=pltpu.MemorySpace.HBM)
    out = pl.pallas_call(
        _mlp_kernel,
        out_shape=jax.ShapeDtypeStruct((1, B), jnp.float32),
        in_specs=[hbm] * 5 + [pl.BlockSpec(memory_space=pltpu.SMEM)],
        scratch_shapes=[
            pltpu.VMEM((1, 64), jnp.float32),
            pltpu.VMEM((1, 64), jnp.float32),
            pltpu.VMEM((64, 64), jnp.float32),
            pltpu.VMEM((1, 64), jnp.float32),
            pltpu.VMEM((1, 64), jnp.float32),
            pltpu.SemaphoreType.DMA,
        ],
    )(
        W1.reshape(1, -1),
        b1.reshape(1, -1),
        W2,
        b2.reshape(1, -1),
        W3.reshape(1, -1),
        b3,
    )
    return out.reshape(B)


# speedup vs baseline: 1.6350x; 1.0410x over previous
import jax
import jax.numpy as jnp
from jax.experimental import pallas as pl
from jax.experimental.pallas import tpu as pltpu

_RADIUS = 1.0
_MAX_K = 64
_MIN_D = 20.0
_MAX_D = 60.0


def _mlp_kernel(w2_ref, w1_ref, b1_ref, b2_ref, w3_ref, b3_smem, out_ref):
    vol = 4.0 / 3.0 * 3.14159 * _RADIUS**3
    d_mean = jnp.float32(_MAX_K) / jnp.float32(vol)

    h1 = jnp.maximum(d_mean * w1_ref[...] + b1_ref[...], 0.0)
    h2 = jax.lax.dot_general(
        h1, w2_ref[...], (((1,), (1,)), ((), ())),
        preferred_element_type=jnp.float32)
    h2 = jnp.maximum(h2 + b2_ref[...], 0.0)
    z = jnp.sum(h2 * w3_ref[...], axis=-1, keepdims=True) + b3_smem[0]
    t = jax.nn.sigmoid(z)
    thr = _MIN_D + (_MAX_D - _MIN_D) * t
    out_ref[...] = jnp.broadcast_to(thr, out_ref.shape)


def kernel(xyz, W1, b1, W2, b2, W3, b3):
    B = xyz.shape[0]
    out = pl.pallas_call(
        _mlp_kernel,
        out_shape=jax.ShapeDtypeStruct((1, B), jnp.float32),
        in_specs=[pl.BlockSpec(memory_space=pltpu.MemorySpace.VMEM)] * 5
        + [pl.BlockSpec(memory_space=pltpu.SMEM)],
    )(
        W2,
        W1.reshape(1, -1),
        b1.reshape(1, -1),
        b2.reshape(1, -1),
        W3.reshape(1, -1),
        b3,
    )
    return out.reshape(B)
